# baseline (speedup 1.0000x reference)
import functools

import jax
import jax.numpy as jnp
from jax import lax
from jax.experimental import pallas as pl
from jax.experimental.pallas import tpu as pltpu
from jax.experimental.pallas import tpu_sc as plsc

N = 10000
NP = 10240
E = 320000
D = 128
NG = 16
NCLS = 3
NSUB = 16
RPT = NP // NSUB
EPT = E // NSUB
C = 80
NCH = EPT // C
NCHP = 256
NB = 16
NSLAB = NCHP // NB
R = 2048
GRID = NP // R

f32 = jnp.float32


def _deg_body(s3, d3, deg_in, deg_out, idx, ones, zbuf, deg_sp):
    cid = lax.axis_index("c")
    tid = lax.axis_index("s")
    base = tid * RPT

    for k in range(C // 16):
        ones[pl.ds(16 * k, 16)] = jnp.ones((16,), f32)
    for k in range(RPT // 16):
        zbuf[pl.ds(16 * k, 16)] = jnp.zeros((16,), f32)

    pltpu.sync_copy(zbuf, deg_sp.at[pl.ds(base, RPT)])
    plsc.subcore_barrier()

    def run(e3, out):
        pltpu.sync_copy(e3.at[tid], idx)

        def body(j, carry):
            pltpu.sync_copy(ones, deg_sp.at[idx.at[j]], add=True)
            return carry

        lax.fori_loop(0, NCHP, body, 0)
        plsc.subcore_barrier()
        pltpu.sync_copy(deg_sp.at[pl.ds(base, RPT)], out.at[pl.ds(base, RPT)])

    @pl.when(cid == 0)
    def _():
        run(d3, deg_in)

    @pl.when(cid == 1)
    def _():
        run(s3, deg_out)


@functools.cache
def _deg_call():
    mesh = plsc.VectorSubcoreMesh(core_axis_name="c", subcore_axis_name="s")
    return pl.kernel(
        _deg_body,
        out_type=(jax.ShapeDtypeStruct((NP,), f32),
                  jax.ShapeDtypeStruct((NP,), f32)),
        mesh=mesh,
        scratch_types=[
            pltpu.VMEM((NCHP, C), jnp.int32),
            pltpu.VMEM((C,), f32),
            pltpu.VMEM((RPT,), f32),
            pltpu.VMEM_SHARED((NP,), f32),
        ],
    )


def _agg_body(xin, xout, s3, d3, out_in, out_out, gidx, sidx,
              rows0, rows1, acc, sem0, sem1):
    cid = lax.axis_index("c")
    tid = lax.axis_index("s")
    base = tid * RPT
    rows = (rows0, rows1)
    sems = (sem0, sem1)

    def run(x, g3, c3, out):
        pltpu.sync_copy(x.at[pl.ds(base, RPT)], acc.at[pl.ds(base, RPT)])
        plsc.subcore_barrier()

        def slab(si, carry):
            pltpu.sync_copy(g3.at[tid].at[pl.ds(si * NB, NB)], gidx)
            pltpu.sync_copy(c3.at[tid].at[pl.ds(si * NB, NB)], sidx)
            def body(j, c2):
                pltpu.async_copy(x.at[gidx.at[j]], rows[0], sems[0]).wait()
                pltpu.sync_copy(rows[0], acc.at[sidx.at[j]], add=True)
                return c2

            lax.fori_loop(0, NB, body, 0)
            return carry

        lax.fori_loop(0, NSLAB, slab, 0)
        plsc.subcore_barrier()
        pltpu.sync_copy(acc.at[pl.ds(base, RPT)], out.at[pl.ds(base, RPT)])

    @pl.when(cid == 0)
    def _():
        run(xin, s3, d3, out_in)

    @pl.when(cid == 1)
    def _():
        run(xout, d3, s3, out_out)


@functools.cache
def _agg_call():
    mesh = plsc.VectorSubcoreMesh(core_axis_name="c", subcore_axis_name="s")
    return pl.kernel(
        _agg_body,
        out_type=(jax.ShapeDtypeStruct((NP, D), f32),
                  jax.ShapeDtypeStruct((NP, D), f32)),
        mesh=mesh,
        scratch_types=[
            pltpu.VMEM((NB, C), jnp.int32),
            pltpu.VMEM((NB, C), jnp.int32),
            pltpu.VMEM((C, D), f32),
            pltpu.VMEM((C, D), f32),
            pltpu.VMEM_SHARED((NP, D), f32),
            pltpu.SemaphoreType.DMA,
            pltpu.SemaphoreType.DMA,
        ],
    )


def _pre_body(x_ref, degi_ref, dego_ref, wi_ref, wo_ref, yin_ref, yout_ref):
    disi = lax.rsqrt(degi_ref[...] + 1.0)
    diso = lax.rsqrt(dego_ref[...] + 1.0)
    x = x_ref[...]
    yin_ref[...] = disi * jnp.dot(x, wi_ref[...], preferred_element_type=f32)
    yout_ref[...] = diso * jnp.dot(x, wo_ref[...], preferred_element_type=f32)


def _mid_body(si_ref, so_ref, degi_ref, dego_ref, bi_ref, bo_ref,
              wi_ref, wo_ref, yin_ref, yout_ref):
    disi = lax.rsqrt(degi_ref[...] + 1.0)
    diso = lax.rsqrt(dego_ref[...] + 1.0)
    beta = 0.5 * (bi_ref[...] + bo_ref[...])
    h = 0.5 * (disi * si_ref[...] + diso * so_ref[...]) + beta
    h = jnp.maximum(h, 0.0)
    yin_ref[...] = disi * jnp.dot(h, wi_ref[...], preferred_element_type=f32)
    yout_ref[...] = diso * jnp.dot(h, wo_ref[...], preferred_element_type=f32)


def _fin_body(si_ref, so_ref, degi_ref, dego_ref, bi_ref, bo_ref, batch_ref,
              wp1_ref, bp1_ref, wp2_ref, bp2_ref, out_ref, acc, cnt):
    i = pl.program_id(0)

    @pl.when(i == 0)
    def _():
        acc[...] = jnp.zeros_like(acc)
        cnt[...] = jnp.zeros_like(cnt)

    disi = lax.rsqrt(degi_ref[...] + 1.0)
    diso = lax.rsqrt(dego_ref[...] + 1.0)
    beta = 0.5 * (bi_ref[...] + bo_ref[...])
    h = 0.5 * (disi * si_ref[...] + diso * so_ref[...]) + beta

    onehot = (batch_ref[...] ==
              lax.broadcasted_iota(jnp.int32, (1, NG), 1)).astype(f32)
    dn = (((0,), (0,)), ((), ()))
    acc[...] += lax.dot_general(onehot, h, dn, preferred_element_type=f32)
    cnt[...] += lax.dot_general(onehot, jnp.ones_like(h), dn,
                                preferred_element_type=f32)

    @pl.when(i == GRID - 1)
    def _():
        pooled = acc[...] / jnp.maximum(cnt[...], 1.0)
        t = jnp.dot(pooled, wp1_ref[...], preferred_element_type=f32)
        t = jnp.maximum(t + bp1_ref[...], 0.0)
        out_ref[...] = jnp.dot(t, wp2_ref[...],
                               preferred_element_type=f32) + bp2_ref[...]


def _row_spec(last):
    return pl.BlockSpec((R, last), lambda i: (i, 0))


def _full_spec(a, b):
    return pl.BlockSpec((a, b), lambda i: (0, 0))


_pre_call = pl.pallas_call(
    _pre_body,
    grid=(GRID,),
    in_specs=[_row_spec(D), _row_spec(1), _row_spec(1),
              _full_spec(D, D), _full_spec(D, D)],
    out_specs=[_row_spec(D), _row_spec(D)],
    out_shape=[jax.ShapeDtypeStruct((NP, D), f32)] * 2,
)

_mid_call = pl.pallas_call(
    _mid_body,
    grid=(GRID,),
    in_specs=[_row_spec(D), _row_spec(D), _row_spec(1), _row_spec(1),
              _full_spec(1, D), _full_spec(1, D),
              _full_spec(D, D), _full_spec(D, D)],
    out_specs=[_row_spec(D), _row_spec(D)],
    out_shape=[jax.ShapeDtypeStruct((NP, D), f32)] * 2,
)

_fin_call = pl.pallas_call(
    _fin_body,
    grid=(GRID,),
    in_specs=[_row_spec(D), _row_spec(D), _row_spec(1), _row_spec(1),
              _full_spec(1, D), _full_spec(1, D), _row_spec(1),
              _full_spec(D, D), _full_spec(1, D),
              _full_spec(D, D), _full_spec(1, D)],
    out_specs=_full_spec(NG, D),
    out_shape=jax.ShapeDtypeStruct((NG, D), f32),
    scratch_shapes=[pltpu.VMEM((NG, D), f32), pltpu.VMEM((NG, D), f32)],
)


def kernel(x, edge_index, batch, W1_in, b1_in, W1_out, b1_out,
           W2_in, b2_in, W2_out, b2_out, W3_in, b3_in, W3_out, b3_out,
           Wp1, bp1, Wp2, bp2):
    xp = jnp.pad(x, ((0, NP - N), (0, 0)))
    s3 = jnp.pad(edge_index[0].reshape(NSUB, NCH, C),
                 ((0, 0), (0, NCHP - NCH), (0, 0)), constant_values=N)
    d3 = jnp.pad(edge_index[1].reshape(NSUB, NCH, C),
                 ((0, 0), (0, NCHP - NCH), (0, 0)), constant_values=N)
    batch2 = jnp.pad(batch, (0, NP - N), constant_values=NG).reshape(NP, 1)
    wp2p = jnp.pad(Wp2, ((0, 0), (0, D - NCLS)))
    bp2p = jnp.pad(bp2, (0, D - NCLS)).reshape(1, D)

    deg_in, deg_out = _deg_call()(s3, d3)
    agg = _agg_call()
    degi2 = deg_in.reshape(NP, 1)
    dego2 = deg_out.reshape(NP, 1)

    yin, yout = _pre_call(xp, degi2, dego2, W1_in, W1_out)
    si, so = agg(yin, yout, s3, d3)
    yin, yout = _mid_call(si, so, degi2, dego2,
                          b1_in.reshape(1, D), b1_out.reshape(1, D),
                          W2_in, W2_out)
    si, so = agg(yin, yout, s3, d3)
    yin, yout = _mid_call(si, so, degi2, dego2,
                          b2_in.reshape(1, D), b2_out.reshape(1, D),
                          W3_in, W3_out)
    si, so = agg(yin, yout, s3, d3)
    out = _fin_call(si, so, degi2, dego2,
                    b3_in.reshape(1, D), b3_out.reshape(1, D), batch2,
                    Wp1, bp1.reshape(1, D), wp2p, bp2p)
    return out[:, :NCLS]


# speedup vs baseline: 2.5061x; 2.5061x over previous
import functools

import jax
import jax.numpy as jnp
from jax import lax
from jax.experimental import pallas as pl
from jax.experimental.pallas import tpu as pltpu
from jax.experimental.pallas import tpu_sc as plsc

N = 10000
NP = 10240
E = 320000
D = 128
NG = 16
NCLS = 3
NSUB = 16
RPT = NP // NSUB
EPT = E // NSUB
C = 80
NCH = EPT // C
NCHP = 256
NB = 16
NSLAB = NCHP // NB
R = 2048
GRID = NP // R

f32 = jnp.float32


def _deg_body(s3, d3, deg_in, deg_out, idx, ones, zbuf, deg_sp):
    cid = lax.axis_index("c")
    tid = lax.axis_index("s")
    base = tid * RPT

    for k in range(C // 16):
        ones[pl.ds(16 * k, 16)] = jnp.ones((16,), f32)
    for k in range(RPT // 16):
        zbuf[pl.ds(16 * k, 16)] = jnp.zeros((16,), f32)

    pltpu.sync_copy(zbuf, deg_sp.at[pl.ds(base, RPT)])
    plsc.subcore_barrier()

    def run(e3, out):
        pltpu.sync_copy(e3.at[tid], idx)

        def body(j, carry):
            pltpu.sync_copy(ones, deg_sp.at[idx.at[j]], add=True)
            return carry

        lax.fori_loop(0, NCH, body, 0)
        plsc.subcore_barrier()
        pltpu.sync_copy(deg_sp.at[pl.ds(base, RPT)], out.at[pl.ds(base, RPT)])

    @pl.when(cid == 0)
    def _():
        run(d3, deg_in)

    @pl.when(cid == 1)
    def _():
        run(s3, deg_out)


@functools.cache
def _deg_call():
    mesh = plsc.VectorSubcoreMesh(core_axis_name="c", subcore_axis_name="s")
    return pl.kernel(
        _deg_body,
        out_type=(jax.ShapeDtypeStruct((NP,), f32),
                  jax.ShapeDtypeStruct((NP,), f32)),
        mesh=mesh,
        scratch_types=[
            pltpu.VMEM((NCHP, C), jnp.int32),
            pltpu.VMEM((C,), f32),
            pltpu.VMEM((RPT,), f32),
            pltpu.VMEM_SHARED((NP,), f32),
        ],
    )


def _agg_body(xin, xout, s3, d3, out_in, out_out, gidx, sidx,
              rows0, rows1, acc, sem0, sem1):
    cid = lax.axis_index("c")
    tid = lax.axis_index("s")
    base = tid * RPT
    rows = (rows0, rows1)
    sems = (sem0, sem1)

    def run(x, g3, c3, out):
        pltpu.sync_copy(x.at[pl.ds(base, RPT)], acc.at[pl.ds(base, RPT)])
        plsc.subcore_barrier()

        def slab(si, carry):
            pltpu.sync_copy(g3.at[tid].at[pl.ds(si * NB, NB)], gidx)
            pltpu.sync_copy(c3.at[tid].at[pl.ds(si * NB, NB)], sidx)
            n = jnp.minimum(NB, NCH - si * NB)
            pltpu.async_copy(x.at[gidx.at[0]], rows[0], sems[0])

            def pair(p, c2):
                j0 = 2 * p
                j1 = j0 + 1
                cp1 = pltpu.async_copy(x.at[gidx.at[j1]], rows[1], sems[1])
                pltpu.make_async_copy(x.at[gidx.at[j0]], rows[0],
                                      sems[0]).wait()
                pltpu.sync_copy(rows[0], acc.at[sidx.at[j0]], add=True)

                @pl.when(j1 + 1 < n)
                def _():
                    pltpu.async_copy(x.at[gidx.at[j1 + 1]], rows[0], sems[0])

                cp1.wait()
                pltpu.sync_copy(rows[1], acc.at[sidx.at[j1]], add=True)
                return c2

            lax.fori_loop(0, n // 2, pair, 0)
            return carry

        lax.fori_loop(0, NSLAB, slab, 0)
        plsc.subcore_barrier()
        pltpu.sync_copy(acc.at[pl.ds(base, RPT)], out.at[pl.ds(base, RPT)])

    @pl.when(cid == 0)
    def _():
        run(xin, s3, d3, out_in)

    @pl.when(cid == 1)
    def _():
        run(xout, d3, s3, out_out)


@functools.cache
def _agg_call():
    mesh = plsc.VectorSubcoreMesh(core_axis_name="c", subcore_axis_name="s")
    return pl.kernel(
        _agg_body,
        out_type=(jax.ShapeDtypeStruct((NP, D), f32),
                  jax.ShapeDtypeStruct((NP, D), f32)),
        mesh=mesh,
        scratch_types=[
            pltpu.VMEM((NB, C), jnp.int32),
            pltpu.VMEM((NB, C), jnp.int32),
            pltpu.VMEM((C, D), f32),
            pltpu.VMEM((C, D), f32),
            pltpu.VMEM_SHARED((NP, D), f32),
            pltpu.SemaphoreType.DMA,
            pltpu.SemaphoreType.DMA,
        ],
    )


def _pre_body(x_ref, degi_ref, dego_ref, wi_ref, wo_ref, yin_ref, yout_ref):
    disi = lax.rsqrt(degi_ref[...] + 1.0)
    diso = lax.rsqrt(dego_ref[...] + 1.0)
    x = x_ref[...]
    yin_ref[...] = disi * jnp.dot(x, wi_ref[...], preferred_element_type=f32)
    yout_ref[...] = diso * jnp.dot(x, wo_ref[...], preferred_element_type=f32)


def _mid_body(si_ref, so_ref, degi_ref, dego_ref, bi_ref, bo_ref,
              wi_ref, wo_ref, yin_ref, yout_ref):
    disi = lax.rsqrt(degi_ref[...] + 1.0)
    diso = lax.rsqrt(dego_ref[...] + 1.0)
    beta = 0.5 * (bi_ref[...] + bo_ref[...])
    h = 0.5 * (disi * si_ref[...] + diso * so_ref[...]) + beta
    h = jnp.maximum(h, 0.0)
    yin_ref[...] = disi * jnp.dot(h, wi_ref[...], preferred_element_type=f32)
    yout_ref[...] = diso * jnp.dot(h, wo_ref[...], preferred_element_type=f32)


def _fin_body(si_ref, so_ref, degi_ref, dego_ref, bi_ref, bo_ref, batch_ref,
              wp1_ref, bp1_ref, wp2_ref, bp2_ref, out_ref, acc, cnt):
    i = pl.program_id(0)

    @pl.when(i == 0)
    def _():
        acc[...] = jnp.zeros_like(acc)
        cnt[...] = jnp.zeros_like(cnt)

    disi = lax.rsqrt(degi_ref[...] + 1.0)
    diso = lax.rsqrt(dego_ref[...] + 1.0)
    beta = 0.5 * (bi_ref[...] + bo_ref[...])
    h = 0.5 * (disi * si_ref[...] + diso * so_ref[...]) + beta

    onehot = (batch_ref[...] ==
              lax.broadcasted_iota(jnp.int32, (1, NG), 1)).astype(f32)
    dn = (((0,), (0,)), ((), ()))
    acc[...] += lax.dot_general(onehot, h, dn, preferred_element_type=f32)
    cnt[...] += lax.dot_general(onehot, jnp.ones_like(h), dn,
                                preferred_element_type=f32)

    @pl.when(i == GRID - 1)
    def _():
        pooled = acc[...] / jnp.maximum(cnt[...], 1.0)
        t = jnp.dot(pooled, wp1_ref[...], preferred_element_type=f32)
        t = jnp.maximum(t + bp1_ref[...], 0.0)
        out_ref[...] = jnp.dot(t, wp2_ref[...],
                               preferred_element_type=f32) + bp2_ref[...]


def _row_spec(last):
    return pl.BlockSpec((R, last), lambda i: (i, 0))


def _full_spec(a, b):
    return pl.BlockSpec((a, b), lambda i: (0, 0))


_pre_call = pl.pallas_call(
    _pre_body,
    grid=(GRID,),
    in_specs=[_row_spec(D), _row_spec(1), _row_spec(1),
              _full_spec(D, D), _full_spec(D, D)],
    out_specs=[_row_spec(D), _row_spec(D)],
    out_shape=[jax.ShapeDtypeStruct((NP, D), f32)] * 2,
)

_mid_call = pl.pallas_call(
    _mid_body,
    grid=(GRID,),
    in_specs=[_row_spec(D), _row_spec(D), _row_spec(1), _row_spec(1),
              _full_spec(1, D), _full_spec(1, D),
              _full_spec(D, D), _full_spec(D, D)],
    out_specs=[_row_spec(D), _row_spec(D)],
    out_shape=[jax.ShapeDtypeStruct((NP, D), f32)] * 2,
)

_fin_call = pl.pallas_call(
    _fin_body,
    grid=(GRID,),
    in_specs=[_row_spec(D), _row_spec(D), _row_spec(1), _row_spec(1),
              _full_spec(1, D), _full_spec(1, D), _row_spec(1),
              _full_spec(D, D), _full_spec(1, D),
              _full_spec(D, D), _full_spec(1, D)],
    out_specs=_full_spec(NG, D),
    out_shape=jax.ShapeDtypeStruct((NG, D), f32),
    scratch_shapes=[pltpu.VMEM((NG, D), f32), pltpu.VMEM((NG, D), f32)],
)


def kernel(x, edge_index, batch, W1_in, b1_in, W1_out, b1_out,
           W2_in, b2_in, W2_out, b2_out, W3_in, b3_in, W3_out, b3_out,
           Wp1, bp1, Wp2, bp2):
    xp = jnp.pad(x, ((0, NP - N), (0, 0)))
    s3 = jnp.pad(edge_index[0].reshape(NSUB, NCH, C),
                 ((0, 0), (0, NCHP - NCH), (0, 0)), constant_values=N)
    d3 = jnp.pad(edge_index[1].reshape(NSUB, NCH, C),
                 ((0, 0), (0, NCHP - NCH), (0, 0)), constant_values=N)
    batch2 = jnp.pad(batch, (0, NP - N), constant_values=NG).reshape(NP, 1)
    wp2p = jnp.pad(Wp2, ((0, 0), (0, D - NCLS)))
    bp2p = jnp.pad(bp2, (0, D - NCLS)).reshape(1, D)

    deg_in, deg_out = _deg_call()(s3, d3)
    agg = _agg_call()
    degi2 = deg_in.reshape(NP, 1)
    dego2 = deg_out.reshape(NP, 1)

    yin, yout = _pre_call(xp, degi2, dego2, W1_in, W1_out)
    si, so = agg(yin, yout, s3, d3)
    yin, yout = _mid_call(si, so, degi2, dego2,
                          b1_in.reshape(1, D), b1_out.reshape(1, D),
                          W2_in, W2_out)
    si, so = agg(yin, yout, s3, d3)
    yin, yout = _mid_call(si, so, degi2, dego2,
                          b2_in.reshape(1, D), b2_out.reshape(1, D),
                          W3_in, W3_out)
    si, so = agg(yin, yout, s3, d3)
    out = _fin_call(si, so, degi2, dego2,
                    b3_in.reshape(1, D), b3_out.reshape(1, D), batch2,
                    Wp1, bp1.reshape(1, D), wp2p, bp2p)
    return out[:, :NCLS]
